# Initial kernel scaffold; baseline (speedup 1.0000x reference)
import jax
import jax.numpy as jnp
from jax.experimental import pallas as pl


def kernel(inters, rels, labels, rels_label, multilab_weights):
    raise NotImplementedError("write your pallas kernel here")


# speedup vs baseline: 1.9351x; 1.9351x over previous
import functools

import jax
import jax.numpy as jnp
from jax.experimental import pallas as pl
from jax.experimental.pallas import tpu as pltpu

_MARGIN = 1.0
_LYMBDA = 1.0
_N_RELS = 40

_BR = 1024


def _body(x_ref, w_ref, lab_ref, r_ref, rlab_ref, out_ref, acc_ref, *, batch):
    step = pl.program_id(0)
    nsteps = pl.num_programs(0)

    @pl.when(step == 0)
    def _():
        acc_ref[0] = 0.0
        acc_ref[1] = 0.0
        acc_ref[2] = 0.0

    x = x_ref[...]
    wf = w_ref[...].astype(jnp.float32)
    lab = lab_ref[...]
    br, c = x.shape
    col = jax.lax.broadcasted_iota(jnp.int32, (br, c), 1)
    tmask = (col == lab).astype(jnp.float32)
    s = jax.nn.sigmoid(x)
    row_ws = jnp.sum(wf * s, axis=1)
    row_w = jnp.sum(wf, axis=1)
    pos = jnp.sum(s * tmask, axis=1)
    wt = jnp.sum(wf * tmask, axis=1)
    row_loss = (_MARGIN - pos) * (row_w - wt) + (row_ws - wt * pos)
    acc_ref[0] += jnp.sum(row_loss)

    r = r_ref[...]
    rlab = rlab_ref[...]
    rc = r.shape[1]
    colr = jax.lax.broadcasted_iota(jnp.int32, (br, rc), 1)
    rmask = (colr == rlab).astype(jnp.float32)
    rs = jax.nn.sigmoid(r)
    posr = jnp.sum(rs * rmask, axis=1)
    row_rs = jnp.sum(rs, axis=1)
    valid = (rlab[:, 0] != _N_RELS).astype(jnp.float32)
    per_row = (_MARGIN - posr) * (rc - 1.0) + (row_rs - posr)
    acc_ref[1] += jnp.sum(per_row * valid)
    acc_ref[2] += jnp.sum(valid)

    @pl.when(step == nsteps - 1)
    def _():
        part1 = _LYMBDA * acc_ref[0] / batch
        cnt = acc_ref[2]
        part2 = jnp.where(cnt > 0.0, acc_ref[1] / jnp.maximum(cnt, 1.0), 0.0)
        out_ref[...] = jnp.full((1, 1), part1 + part2, dtype=jnp.float32)


@jax.jit
def kernel(inters, rels, labels, rels_label, multilab_weights):
    batch, n_classes = inters.shape
    rc = rels.shape[1]
    rlab2 = rels_label.reshape(batch, 1)
    grid = batch // _BR
    out = pl.pallas_call(
        functools.partial(_body, batch=batch),
        grid=(grid,),
        in_specs=[
            pl.BlockSpec((_BR, n_classes), lambda i: (i, 0)),
            pl.BlockSpec((_BR, n_classes), lambda i: (i, 0)),
            pl.BlockSpec((_BR, 1), lambda i: (i, 0)),
            pl.BlockSpec((_BR, rc), lambda i: (i, 0)),
            pl.BlockSpec((_BR, 1), lambda i: (i, 0)),
        ],
        out_specs=pl.BlockSpec((1, 1), lambda i: (0, 0)),
        out_shape=jax.ShapeDtypeStruct((1, 1), jnp.float32),
        scratch_shapes=[pltpu.SMEM((4,), jnp.float32)],
    )(inters, multilab_weights, labels, rels, rlab2)
    return out.reshape(1)
